# baseline (speedup 1.0000x reference)
import functools
import math

import jax
import jax.numpy as jnp
from jax import lax
from jax.experimental import pallas as pl
from jax.experimental.pallas import tpu as pltpu
from jax.experimental.pallas import tpu_sc as plsc

H = 16
TOPK = 2


def _gelu(v):
    return 0.5 * v * (1.0 + jax.lax.erf(v * (1.0 / math.sqrt(2.0))))


def _router_body(nk, x_ref, wgt_ref, bg_ref, lg_ref, acc_ref):
    k = pl.program_id(0)

    @pl.when(k == 0)
    def _init():
        acc_ref[...] = jnp.zeros_like(acc_ref)

    nb, sr, d = x_ref.shape
    ne = wgt_ref.shape[0]
    tots = [None] * nb
    for s in range(sr):
        w_sl = wgt_ref[:, s * d:(s + 1) * d]
        for b in range(nb):
            xs = jnp.broadcast_to(x_ref[b, s:s + 1, :], (ne, d))
            p = w_sl * xs
            tots[b] = p if tots[b] is None else tots[b] + p
    for b in range(nb):
        acc_ref[b] += tots[b]

    @pl.when(k == nk - 1)
    def _finish():
        logits = jnp.sum(acc_ref[...], axis=2) + bg_ref[...]
        lg_ref[...] = jnp.concatenate([logits[0:1, :], logits[1:2, :]], axis=1)


def _sc_take(v, idx):
    return v.at[idx].get(mode="promise_in_bounds")


def _sc_group_reduce(v, op, iota):
    for k in (1, 2, 4):
        v = op(v, _sc_take(v, jnp.bitwise_xor(iota, k)))
    return v


def _route_body(lg_hbm, ti_hbm, ga_hbm, v_scr, i_scr, g_scr):
    wid = lax.axis_index("s") * 2 + lax.axis_index("c")

    @pl.when(wid == 0)
    def _():
        pltpu.sync_copy(lg_hbm, v_scr)
        lg = v_scr[...]
        iota = lax.iota(jnp.int32, 16)
        lane_e = jnp.bitwise_and(iota, 7)
        m = _sc_group_reduce(lg, jnp.maximum, iota)
        p = jnp.exp(lg - m)
        s = _sc_group_reduce(p, jnp.add, iota)
        p = p / s
        big = jnp.full((16,), 8, jnp.int32)
        m1 = _sc_group_reduce(p, jnp.maximum, iota)
        i1 = _sc_group_reduce(jnp.where(p == m1, lane_e, big), jnp.minimum, iota)
        p2 = jnp.where(lane_e == i1, jnp.full((16,), -1.0, jnp.float32), p)
        m2 = _sc_group_reduce(p2, jnp.maximum, iota)
        i2 = _sc_group_reduce(jnp.where(p2 == m2, lane_e, big), jnp.minimum, iota)
        src = jnp.bitwise_and(jnp.left_shift(jnp.right_shift(iota, 1), 3), 15)
        odd = jnp.bitwise_and(iota, 1) == 1
        i_scr[...] = jnp.where(odd, _sc_take(i2, src), _sc_take(i1, src))
        g_scr[...] = jnp.where(odd, _sc_take(m2, src), _sc_take(m1, src))
        pltpu.sync_copy(i_scr, ti_hbm)
        pltpu.sync_copy(g_scr, ga_hbm)


def _route_sc(logits16):
    mesh = plsc.VectorSubcoreMesh(core_axis_name="c", subcore_axis_name="s")
    return pl.kernel(
        _route_body,
        mesh=mesh,
        out_type=[
            jax.ShapeDtypeStruct((16,), jnp.int32),
            jax.ShapeDtypeStruct((16,), jnp.float32),
        ],
        scratch_types=[
            pltpu.VMEM((16,), jnp.float32),
            pltpu.VMEM((16,), jnp.int32),
            pltpu.VMEM((16,), jnp.float32),
        ],
    )(logits16)


def _moe_body(topi_ref, gates_ref, x_ref, w1a_ref, w1b_ref, b1a_ref, b1b_ref,
              w2a_ref, w2b_ref, b2a_ref, b2b_ref, w3_ref, b3_ref, y_ref,
              out_ref, w3bf_ref):
    b = pl.program_id(0)
    t = pl.program_id(1)

    @pl.when(jnp.logical_and(b == 0, t == 0))
    def _cvt():
        w3bf_ref[...] = w3_ref[...].astype(jnp.bfloat16)

    p = lax.rem(t, 2)
    g0 = gates_ref[2 * b]
    g1 = gates_ref[2 * b + 1]
    xt = x_ref[0]
    xbf = xt.astype(jnp.bfloat16)
    wa = jnp.concatenate([w1a_ref[0], w1b_ref[0]], axis=1)
    ba = jnp.concatenate([b1a_ref[0, 0], b1b_ref[0, 0]], axis=0)
    wb = jnp.concatenate([g0 * w2a_ref[0], g1 * w2b_ref[0]], axis=0)
    wb = wb.astype(jnp.bfloat16)
    bb = g0 * b2a_ref[0, 0] + g1 * b2b_ref[0, 0]
    dh = wa.shape[0]
    y_ref[0] = (jnp.dot(out_ref[1 - p].astype(jnp.bfloat16), w3bf_ref[...],
                        preferred_element_type=jnp.float32) + b3_ref[0])
    for h in range(H):
        sl = slice(h * dh, (h + 1) * dh)
        a = jnp.dot(xbf[:, sl], wa, preferred_element_type=jnp.float32) + ba
        ag = _gelu(a).astype(jnp.bfloat16)
        mo = jnp.dot(ag, wb, preferred_element_type=jnp.float32) + bb
        out_ref[p, :, sl] = xt[:, sl] + mo


def kernel(x, Wg, bg, W1, b1, W2, b2, W3, b3):
    B, S, D = x.shape
    E = Wg.shape[1]
    DH = W1.shape[1]
    DHH = W1.shape[2]

    wg_t = Wg.T
    SR = 64
    NK = S // SR

    lg = pl.pallas_call(
        functools.partial(_router_body, NK),
        grid=(NK,),
        in_specs=[
            pl.BlockSpec((B, SR, D), lambda k: (0, k, 0)),
            pl.BlockSpec((E, SR * D), lambda k: (0, k)),
            pl.BlockSpec((1, E), lambda k: (0, 0)),
        ],
        out_specs=pl.BlockSpec((1, B * E), lambda k: (0, 0)),
        out_shape=jax.ShapeDtypeStruct((1, B * E), jnp.float32),
        scratch_shapes=[pltpu.VMEM((B, E, D), jnp.float32)],
    )(x, wg_t, bg.reshape(1, E))

    topi, gates = _route_sc(lg.reshape(B * E))

    TS = 512
    T = S // TS
    w1_bf = W1.astype(jnp.bfloat16)
    b1r = b1.reshape(E, 1, DHH)
    b2r = b2.reshape(E, 1, DH)
    b3r = b3.reshape(1, D)

    grid_spec = pltpu.PrefetchScalarGridSpec(
        num_scalar_prefetch=2,
        grid=(B, T + 1),
        in_specs=[
            pl.BlockSpec((1, TS, D),
                         lambda b, t, ti, ga: (b, jnp.minimum(t, T - 1), 0)),
            pl.BlockSpec((1, DH, DHH), lambda b, t, ti, ga: (ti[2 * b], 0, 0)),
            pl.BlockSpec((1, DH, DHH), lambda b, t, ti, ga: (ti[2 * b + 1], 0, 0)),
            pl.BlockSpec((1, 1, DHH), lambda b, t, ti, ga: (ti[2 * b], 0, 0)),
            pl.BlockSpec((1, 1, DHH), lambda b, t, ti, ga: (ti[2 * b + 1], 0, 0)),
            pl.BlockSpec((1, DHH, DH), lambda b, t, ti, ga: (ti[2 * b], 0, 0)),
            pl.BlockSpec((1, DHH, DH), lambda b, t, ti, ga: (ti[2 * b + 1], 0, 0)),
            pl.BlockSpec((1, 1, DH), lambda b, t, ti, ga: (ti[2 * b], 0, 0)),
            pl.BlockSpec((1, 1, DH), lambda b, t, ti, ga: (ti[2 * b + 1], 0, 0)),
            pl.BlockSpec((D, D), lambda b, t, ti, ga: (0, 0)),
            pl.BlockSpec((1, D), lambda b, t, ti, ga: (0, 0)),
        ],
        out_specs=pl.BlockSpec((1, TS, D),
                               lambda b, t, ti, ga: (b, jnp.maximum(t - 1, 0), 0)),
        scratch_shapes=[pltpu.VMEM((2, TS, D), jnp.float32),
                        pltpu.VMEM((D, D), jnp.bfloat16)],
    )

    y = pl.pallas_call(
        _moe_body,
        grid_spec=grid_spec,
        out_shape=jax.ShapeDtypeStruct((B, S, D), jnp.float32),
    )(topi, gates, x, w1_bf, w1_bf, b1r, b1r, W2, W2, b2r, b2r, W3, b3r)
    return y


# speedup vs baseline: 1.1488x; 1.1488x over previous
import functools
import math

import jax
import jax.numpy as jnp
from jax import lax
from jax.experimental import pallas as pl
from jax.experimental.pallas import tpu as pltpu
from jax.experimental.pallas import tpu_sc as plsc

H = 16
TOPK = 2


def _gelu(v):
    return 0.5 * v * (1.0 + jax.lax.erf(v * (1.0 / math.sqrt(2.0))))


def _router_body(nk, x_ref, wgt_ref, bg_ref, lg_ref, acc_ref):
    k = pl.program_id(0)

    @pl.when(k == 0)
    def _init():
        acc_ref[...] = jnp.zeros_like(acc_ref)

    nb, sr, d = x_ref.shape
    ne = wgt_ref.shape[0]
    tots = [None] * nb
    for s in range(sr):
        w_sl = wgt_ref[:, s * d:(s + 1) * d]
        for b in range(nb):
            xs = jnp.broadcast_to(x_ref[b, s:s + 1, :], (ne, d))
            p = w_sl * xs
            tots[b] = p if tots[b] is None else tots[b] + p
    for b in range(nb):
        acc_ref[b] += tots[b]

    @pl.when(k == nk - 1)
    def _finish():
        logits = jnp.sum(acc_ref[...], axis=2) + bg_ref[...]
        lg_ref[...] = jnp.concatenate([logits[0:1, :], logits[1:2, :]], axis=1)


def _sc_take(v, idx):
    return v.at[idx].get(mode="promise_in_bounds")


def _sc_group_reduce(v, op, iota):
    for k in (1, 2, 4):
        v = op(v, _sc_take(v, jnp.bitwise_xor(iota, k)))
    return v


def _route_body(lg_hbm, ti_hbm, ga_hbm, v_scr, i_scr, g_scr):
    wid = lax.axis_index("s") * 2 + lax.axis_index("c")

    @pl.when(wid == 0)
    def _():
        pltpu.sync_copy(lg_hbm, v_scr)
        lg = v_scr[...]
        iota = lax.iota(jnp.int32, 16)
        lane_e = jnp.bitwise_and(iota, 7)
        m = _sc_group_reduce(lg, jnp.maximum, iota)
        p = jnp.exp(lg - m)
        s = _sc_group_reduce(p, jnp.add, iota)
        p = p / s
        big = jnp.full((16,), 8, jnp.int32)
        m1 = _sc_group_reduce(p, jnp.maximum, iota)
        i1 = _sc_group_reduce(jnp.where(p == m1, lane_e, big), jnp.minimum, iota)
        p2 = jnp.where(lane_e == i1, jnp.full((16,), -1.0, jnp.float32), p)
        m2 = _sc_group_reduce(p2, jnp.maximum, iota)
        i2 = _sc_group_reduce(jnp.where(p2 == m2, lane_e, big), jnp.minimum, iota)
        src = jnp.bitwise_and(jnp.left_shift(jnp.right_shift(iota, 1), 3), 15)
        odd = jnp.bitwise_and(iota, 1) == 1
        i_scr[...] = jnp.where(odd, _sc_take(i2, src), _sc_take(i1, src))
        g_scr[...] = jnp.where(odd, _sc_take(m2, src), _sc_take(m1, src))
        pltpu.sync_copy(i_scr, ti_hbm)
        pltpu.sync_copy(g_scr, ga_hbm)


def _route_sc(logits16):
    mesh = plsc.VectorSubcoreMesh(core_axis_name="c", subcore_axis_name="s")
    return pl.kernel(
        _route_body,
        mesh=mesh,
        out_type=[
            jax.ShapeDtypeStruct((16,), jnp.int32),
            jax.ShapeDtypeStruct((16,), jnp.float32),
        ],
        scratch_types=[
            pltpu.VMEM((16,), jnp.float32),
            pltpu.VMEM((16,), jnp.int32),
            pltpu.VMEM((16,), jnp.float32),
        ],
    )(logits16)


def _moe_body(topi_ref, gates_ref, x_ref, w1a_ref, w1b_ref, b1a_ref, b1b_ref,
              w2a_ref, w2b_ref, b2a_ref, b2b_ref, w3_ref, b3_ref, y_ref,
              out_ref, w3bf_ref):
    b = pl.program_id(0)
    t = pl.program_id(1)

    @pl.when(jnp.logical_and(b == 0, t == 0))
    def _cvt():
        w3bf_ref[...] = w3_ref[...].astype(jnp.bfloat16)

    g0 = gates_ref[2 * b]
    g1 = gates_ref[2 * b + 1]
    xt = x_ref[0]
    xbf = xt.astype(jnp.bfloat16)
    wa = jnp.concatenate([w1a_ref[0], w1b_ref[0]], axis=1)
    ba = jnp.concatenate([b1a_ref[0, 0], b1b_ref[0, 0]], axis=0)
    wb = jnp.concatenate([g0 * w2a_ref[0], g1 * w2b_ref[0]], axis=0)
    wb = wb.astype(jnp.bfloat16)
    bb = g0 * b2a_ref[0, 0] + g1 * b2b_ref[0, 0]
    dh = wa.shape[0]
    for h in range(H):
        sl = slice(h * dh, (h + 1) * dh)
        a = jnp.dot(xbf[:, sl], wa, preferred_element_type=jnp.float32) + ba
        ag = _gelu(a).astype(jnp.bfloat16)
        mo = jnp.dot(ag, wb, preferred_element_type=jnp.float32) + bb
        out_ref[:, sl] = xt[:, sl] + mo
    y_ref[0] = (jnp.dot(out_ref[...].astype(jnp.bfloat16), w3bf_ref[...],
                        preferred_element_type=jnp.float32) + b3_ref[0])


def kernel(x, Wg, bg, W1, b1, W2, b2, W3, b3):
    B, S, D = x.shape
    E = Wg.shape[1]
    DH = W1.shape[1]
    DHH = W1.shape[2]

    wg_t = Wg.T
    SR = 128
    NK = S // SR

    lg = pl.pallas_call(
        functools.partial(_router_body, NK),
        grid=(NK,),
        in_specs=[
            pl.BlockSpec((B, SR, D), lambda k: (0, k, 0)),
            pl.BlockSpec((E, SR * D), lambda k: (0, k)),
            pl.BlockSpec((1, E), lambda k: (0, 0)),
        ],
        out_specs=pl.BlockSpec((1, B * E), lambda k: (0, 0)),
        out_shape=jax.ShapeDtypeStruct((1, B * E), jnp.float32),
        scratch_shapes=[pltpu.VMEM((B, E, D), jnp.float32)],
    )(x, wg_t, bg.reshape(1, E))

    topi, gates = _route_sc(lg.reshape(B * E))

    TS = 512
    T = S // TS
    w1_bf = W1.astype(jnp.bfloat16)
    b1r = b1.reshape(E, 1, DHH)
    b2r = b2.reshape(E, 1, DH)
    b3r = b3.reshape(1, D)

    grid_spec = pltpu.PrefetchScalarGridSpec(
        num_scalar_prefetch=2,
        grid=(B, T),
        in_specs=[
            pl.BlockSpec((1, TS, D), lambda b, t, ti, ga: (b, t, 0)),
            pl.BlockSpec((1, DH, DHH), lambda b, t, ti, ga: (ti[2 * b], 0, 0)),
            pl.BlockSpec((1, DH, DHH), lambda b, t, ti, ga: (ti[2 * b + 1], 0, 0)),
            pl.BlockSpec((1, 1, DHH), lambda b, t, ti, ga: (ti[2 * b], 0, 0)),
            pl.BlockSpec((1, 1, DHH), lambda b, t, ti, ga: (ti[2 * b + 1], 0, 0)),
            pl.BlockSpec((1, DHH, DH), lambda b, t, ti, ga: (ti[2 * b], 0, 0)),
            pl.BlockSpec((1, DHH, DH), lambda b, t, ti, ga: (ti[2 * b + 1], 0, 0)),
            pl.BlockSpec((1, 1, DH), lambda b, t, ti, ga: (ti[2 * b], 0, 0)),
            pl.BlockSpec((1, 1, DH), lambda b, t, ti, ga: (ti[2 * b + 1], 0, 0)),
            pl.BlockSpec((D, D), lambda b, t, ti, ga: (0, 0)),
            pl.BlockSpec((1, D), lambda b, t, ti, ga: (0, 0)),
        ],
        out_specs=pl.BlockSpec((1, TS, D), lambda b, t, ti, ga: (b, t, 0)),
        scratch_shapes=[pltpu.VMEM((TS, D), jnp.float32),
                        pltpu.VMEM((D, D), jnp.bfloat16)],
    )

    y = pl.pallas_call(
        _moe_body,
        grid_spec=grid_spec,
        out_shape=jax.ShapeDtypeStruct((B, S, D), jnp.float32),
    )(topi, gates, x, w1_bf, w1_bf, b1r, b1r, W2, W2, b2r, b2r, W3, b3r)
    return y
